# baseline (device time: 44646 ns/iter reference)
import jax
import jax.numpy as jnp
from jax import lax
from jax.experimental import pallas as pl
from jax.experimental.pallas import tpu as pltpu

N_DEV = 16

_W_ORDER = [0] + list(range(N_DEV - 1, 0, -1))

_GROUPS = {0: (0, 1), 4: (12, 16), 8: (8, 12), 12: (4, 8), 14: (2, 4), 15: (1, 2)}


def kernel(x, w_mat):
    m_total, k_sh = x.shape
    k_total, n = w_mat.shape
    m_per = m_total // N_DEV

    def body(x_ref, w_ref, out_ref, x_st, x_bf, xrot, w_st, acc,
             send_sems, recv_sems, x_sem, w_sems):
        my_i = lax.axis_index("i")

        def w_dma(k):
            r = _W_ORDER[k]
            src = lax.rem(my_i + r, N_DEV)
            return pltpu.make_async_copy(
                w_ref.at[pl.ds(src * k_sh, k_sh), :],
                w_st.at[pl.ds(r * k_sh, k_sh), :], w_sems.at[k],
            )

        x_dma = pltpu.make_async_copy(x_ref, x_st, x_sem)
        x_dma.start()
        w_dma(0).start()
        w_dma(1).start()

        barrier = pltpu.get_barrier_semaphore()
        for s in range(1, N_DEV):
            dst = lax.rem(my_i + s, N_DEV)
            pl.semaphore_signal(
                barrier, inc=1,
                device_id=(dst,), device_id_type=pl.DeviceIdType.MESH,
            )

        x_dma.wait()
        x_bf[...] = x_st[...].astype(jnp.bfloat16)
        xrot[:, pl.ds(0, k_sh)] = x_bf[pl.ds(my_i * m_per, m_per), :]

        pl.semaphore_wait(barrier, N_DEV - 1)

        def make_send(s):
            dst = lax.rem(my_i + s, N_DEV)
            rel = N_DEV - s
            return pltpu.make_async_remote_copy(
                src_ref=x_bf.at[pl.ds(dst * m_per, m_per), :],
                dst_ref=xrot.at[:, pl.ds(rel * k_sh, k_sh)],
                send_sem=send_sems.at[s - 1],
                recv_sem=recv_sems.at[rel],
                device_id=(dst,),
                device_id_type=pl.DeviceIdType.MESH,
            )

        _WAVES = {0: range(1, 5), 4: range(5, 9), 8: range(9, 13), 12: range(13, 16)}
        sends = {}
        for s in _WAVES[0]:
            sends[s] = make_send(s)
            sends[s].start()

        for k in range(N_DEV):
            if k in (4, 8, 12):
                for s in _WAVES[k - 4]:
                    sends[s].wait_send()
                for s in _WAVES[k]:
                    sends[s] = make_send(s)
                    sends[s].start()
            w_dma(k).wait()
            if k + 2 < N_DEV:
                w_dma(k + 2).start()
            if k >= 1:
                s = k
                rel = N_DEV - s
                recv = pltpu.make_async_remote_copy(
                    src_ref=x_bf.at[pl.ds(0, m_per), :],
                    dst_ref=xrot.at[:, pl.ds(rel * k_sh, k_sh)],
                    send_sem=send_sems.at[s - 1],
                    recv_sem=recv_sems.at[rel],
                    device_id=(lax.rem(my_i - s + N_DEV, N_DEV),),
                    device_id_type=pl.DeviceIdType.MESH,
                )
                recv.wait_recv()
            if k in _GROUPS:
                r0, r1 = _GROUPS[k]
                c0, c1 = r0 * k_sh, r1 * k_sh
                part = jnp.dot(
                    xrot[:, c0:c1],
                    w_st[c0:c1, :].astype(jnp.bfloat16),
                    preferred_element_type=jnp.float32,
                )
                if k == 0:
                    acc[...] = part
                else:
                    acc[...] += part

        for s in _WAVES[12]:
            sends[s].wait_send()

        y = acc[...]
        c = 0.7978845608028654
        out_ref[...] = (
            0.5 * y * (1.0 + jnp.tanh(c * (y + 0.044715 * y * y * y)))
        ).astype(jnp.bfloat16)

    return pl.pallas_call(
        body,
        out_shape=jax.ShapeDtypeStruct((m_per, n), jnp.bfloat16),
        in_specs=[
            pl.BlockSpec(memory_space=pl.ANY),
            pl.BlockSpec(memory_space=pl.ANY),
        ],
        out_specs=pl.BlockSpec(memory_space=pltpu.VMEM),
        scratch_shapes=[
            pltpu.VMEM((m_total, k_sh), jnp.float32),
            pltpu.VMEM((m_total, k_sh), jnp.bfloat16),
            pltpu.VMEM((m_per, k_total), jnp.bfloat16),
            pltpu.VMEM((k_total, n), jnp.float32),
            pltpu.VMEM((m_per, n), jnp.float32),
            pltpu.SemaphoreType.DMA((N_DEV - 1,)),
            pltpu.SemaphoreType.DMA((N_DEV,)),
            pltpu.SemaphoreType.DMA,
            pltpu.SemaphoreType.DMA((N_DEV,)),
        ],
        compiler_params=pltpu.CompilerParams(
            collective_id=0,
            vmem_limit_bytes=100 * 1024 * 1024,
        ),
    )(x, w_mat)


# device time: 35451 ns/iter; 1.2594x vs baseline; 1.2594x over previous
import jax
import jax.numpy as jnp
from jax import lax
from jax.experimental import pallas as pl
from jax.experimental.pallas import tpu as pltpu

N_DEV = 16

_W_ORDER = [0] + list(range(N_DEV - 1, 0, -1))

_GROUPS = {0: (0, 1), 4: (12, 16), 8: (8, 12), 12: (4, 8), 14: (2, 4), 15: (1, 2)}


def kernel(x, w_mat):
    m_total, k_sh = x.shape
    k_total, n = w_mat.shape
    m_per = m_total // N_DEV

    def body(x_ref, w_ref, out_ref, x_st, x_bf, xrot, w_st, acc,
             send_sems, recv_sems, x_sem, w_sems):
        my_i = lax.axis_index("i")

        def w_dma(k):
            r = _W_ORDER[k]
            src = lax.rem(my_i + r, N_DEV)
            return pltpu.make_async_copy(
                w_ref.at[pl.ds(src * k_sh, k_sh), :],
                w_st.at[pl.ds(r * k_sh, k_sh), :], w_sems.at[k],
            )

        x_dma = pltpu.make_async_copy(x_ref, x_st, x_sem)
        x_dma.start()
        w_dma(0).start()
        w_dma(1).start()

        barrier = pltpu.get_barrier_semaphore()
        for s in range(1, N_DEV):
            dst = lax.rem(my_i + s, N_DEV)
            pl.semaphore_signal(
                barrier, inc=1,
                device_id=(dst,), device_id_type=pl.DeviceIdType.MESH,
            )

        x_dma.wait()
        x_bf[...] = x_st[...].astype(jnp.bfloat16)
        xrot[:, pl.ds(0, k_sh)] = x_bf[pl.ds(my_i * m_per, m_per), :]

        pl.semaphore_wait(barrier, N_DEV - 1)

        sends = []
        for s in range(1, N_DEV):
            dst = lax.rem(my_i + s, N_DEV)
            rel = N_DEV - s
            rdma = pltpu.make_async_remote_copy(
                src_ref=x_bf.at[pl.ds(dst * m_per, m_per), :],
                dst_ref=xrot.at[:, pl.ds(rel * k_sh, k_sh)],
                send_sem=send_sems.at[s - 1],
                recv_sem=recv_sems.at[rel],
                device_id=(dst,),
                device_id_type=pl.DeviceIdType.MESH,
            )
            rdma.start()
            sends.append(rdma)

        for k in range(N_DEV):
            w_dma(k).wait()
            if k + 2 < N_DEV:
                w_dma(k + 2).start()
            if k >= 1:
                s = k
                rel = N_DEV - s
                recv = pltpu.make_async_remote_copy(
                    src_ref=x_bf.at[pl.ds(0, m_per), :],
                    dst_ref=xrot.at[:, pl.ds(rel * k_sh, k_sh)],
                    send_sem=send_sems.at[s - 1],
                    recv_sem=recv_sems.at[rel],
                    device_id=(lax.rem(my_i - s + N_DEV, N_DEV),),
                    device_id_type=pl.DeviceIdType.MESH,
                )
                recv.wait_recv()
            if k in _GROUPS:
                r0, r1 = _GROUPS[k]
                c0, c1 = r0 * k_sh, r1 * k_sh
                part = jnp.dot(
                    xrot[:, c0:c1],
                    w_st[c0:c1, :].astype(jnp.bfloat16),
                    preferred_element_type=jnp.float32,
                )
                if k == 0:
                    acc[...] = part
                else:
                    acc[...] += part

        for rdma in sends:
            rdma.wait_send()

        y = acc[...]
        c = 0.7978845608028654
        out_ref[...] = (
            0.5 * y * (1.0 + jnp.tanh(c * (y + 0.044715 * y * y * y)))
        ).astype(jnp.bfloat16)

    return pl.pallas_call(
        body,
        out_shape=jax.ShapeDtypeStruct((m_per, n), jnp.bfloat16),
        in_specs=[
            pl.BlockSpec(memory_space=pl.ANY),
            pl.BlockSpec(memory_space=pl.ANY),
        ],
        out_specs=pl.BlockSpec(memory_space=pltpu.VMEM),
        scratch_shapes=[
            pltpu.VMEM((m_total, k_sh), jnp.float32),
            pltpu.VMEM((m_total, k_sh), jnp.bfloat16),
            pltpu.VMEM((m_per, k_total), jnp.bfloat16),
            pltpu.VMEM((k_total, n), jnp.float32),
            pltpu.VMEM((m_per, n), jnp.float32),
            pltpu.SemaphoreType.DMA((N_DEV - 1,)),
            pltpu.SemaphoreType.DMA((N_DEV,)),
            pltpu.SemaphoreType.DMA,
            pltpu.SemaphoreType.DMA((N_DEV,)),
        ],
        compiler_params=pltpu.CompilerParams(
            collective_id=0,
            vmem_limit_bytes=100 * 1024 * 1024,
        ),
    )(x, w_mat)


# device time: 32397 ns/iter; 1.3781x vs baseline; 1.0943x over previous
import jax
import jax.numpy as jnp
from jax import lax
from jax.experimental import pallas as pl
from jax.experimental.pallas import tpu as pltpu

N_DEV = 16

_W_ORDER = [0] + list(range(N_DEV - 1, 0, -1))

_GROUPS = {0: (0, 1), 4: (12, 16), 8: (8, 12), 12: (4, 8), 14: (2, 4), 15: (1, 2)}


def kernel(x, w_mat):
    m_total, k_sh = x.shape
    k_total, n = w_mat.shape
    m_per = m_total // N_DEV

    def body(x_ref, w_ref, out_ref, x_st, x_bf, xrot, w_st, acc,
             send_sems, recv_sems, x_sem, w_sems):
        my_i = lax.axis_index("i")

        def w_dma(k):
            r = _W_ORDER[k]
            src = lax.rem(my_i + r, N_DEV)
            return pltpu.make_async_copy(
                w_ref.at[pl.ds(src * k_sh, k_sh), :],
                w_st.at[pl.ds(r * k_sh, k_sh), :], w_sems.at[k],
            )

        x_dma = pltpu.make_async_copy(x_ref, x_st, x_sem)
        x_dma.start()
        w_dma(0).start()
        w_dma(1).start()

        barrier = pltpu.get_barrier_semaphore()
        for s in range(1, N_DEV):
            dst = lax.rem(my_i + s, N_DEV)
            pl.semaphore_signal(
                barrier, inc=1,
                device_id=(dst,), device_id_type=pl.DeviceIdType.MESH,
            )

        x_dma.wait()
        x_bf[...] = x_st[...].astype(jnp.bfloat16)
        xrot[:, pl.ds(0, k_sh)] = x_bf[pl.ds(my_i * m_per, m_per), :]

        pl.semaphore_wait(barrier, N_DEV - 1)

        sends = []
        for s in range(1, N_DEV):
            dst = lax.rem(my_i + s, N_DEV)
            rel = N_DEV - s
            rdma = pltpu.make_async_remote_copy(
                src_ref=x_bf.at[pl.ds(dst * m_per, m_per), :],
                dst_ref=xrot.at[:, pl.ds(rel * k_sh, k_sh)],
                send_sem=send_sems.at[s - 1],
                recv_sem=recv_sems.at[rel],
                device_id=(dst,),
                device_id_type=pl.DeviceIdType.MESH,
            )
            rdma.start()
            sends.append(rdma)

        for k in range(2, N_DEV):
            w_dma(k).start()

        for k in range(N_DEV):
            w_dma(k).wait()
            if k >= 1:
                s = k
                rel = N_DEV - s
                recv = pltpu.make_async_remote_copy(
                    src_ref=x_bf.at[pl.ds(0, m_per), :],
                    dst_ref=xrot.at[:, pl.ds(rel * k_sh, k_sh)],
                    send_sem=send_sems.at[s - 1],
                    recv_sem=recv_sems.at[rel],
                    device_id=(lax.rem(my_i - s + N_DEV, N_DEV),),
                    device_id_type=pl.DeviceIdType.MESH,
                )
                recv.wait_recv()
            if k in _GROUPS:
                r0, r1 = _GROUPS[k]
                c0, c1 = r0 * k_sh, r1 * k_sh
                part = jnp.dot(
                    xrot[:, c0:c1],
                    w_st[c0:c1, :].astype(jnp.bfloat16),
                    preferred_element_type=jnp.float32,
                )
                if k == 0:
                    acc[...] = part
                else:
                    acc[...] += part

        for rdma in sends:
            rdma.wait_send()

        y = acc[...]
        c = 0.7978845608028654
        out_ref[...] = (
            0.5 * y * (1.0 + jnp.tanh(c * (y + 0.044715 * y * y * y)))
        ).astype(jnp.bfloat16)

    return pl.pallas_call(
        body,
        out_shape=jax.ShapeDtypeStruct((m_per, n), jnp.bfloat16),
        in_specs=[
            pl.BlockSpec(memory_space=pl.ANY),
            pl.BlockSpec(memory_space=pl.ANY),
        ],
        out_specs=pl.BlockSpec(memory_space=pltpu.VMEM),
        scratch_shapes=[
            pltpu.VMEM((m_total, k_sh), jnp.float32),
            pltpu.VMEM((m_total, k_sh), jnp.bfloat16),
            pltpu.VMEM((m_per, k_total), jnp.bfloat16),
            pltpu.VMEM((k_total, n), jnp.float32),
            pltpu.VMEM((m_per, n), jnp.float32),
            pltpu.SemaphoreType.DMA((N_DEV - 1,)),
            pltpu.SemaphoreType.DMA((N_DEV,)),
            pltpu.SemaphoreType.DMA,
            pltpu.SemaphoreType.DMA((N_DEV,)),
        ],
        compiler_params=pltpu.CompilerParams(
            collective_id=0,
            vmem_limit_bytes=100 * 1024 * 1024,
        ),
    )(x, w_mat)


# device time: 32167 ns/iter; 1.3879x vs baseline; 1.0072x over previous
import jax
import jax.numpy as jnp
from jax import lax
from jax.experimental import pallas as pl
from jax.experimental.pallas import tpu as pltpu

N_DEV = 16

_W_ORDER = [0] + list(range(N_DEV - 1, 0, -1))

_GROUPS = {0: (0, 1), 4: (12, 16), 8: (8, 12), 12: (4, 8), 14: (2, 4), 15: (1, 2)}


def kernel(x, w_mat):
    m_total, k_sh = x.shape
    k_total, n = w_mat.shape
    m_per = m_total // N_DEV

    def body(x_ref, w_ref, out_ref, x_st, x_bf, xrot, w_st, acc,
             send_sems, recv_sems, x_sem, w_sems):
        my_i = lax.axis_index("i")

        def w_dma(k):
            r = _W_ORDER[k]
            src = lax.rem(my_i + r, N_DEV)
            return pltpu.make_async_copy(
                w_ref.at[pl.ds(src * k_sh, k_sh), :],
                w_st.at[pl.ds(r * k_sh, k_sh), :], w_sems.at[k],
            )

        x_dma = pltpu.make_async_copy(x_ref, x_st, x_sem)
        x_dma.start()

        barrier = pltpu.get_barrier_semaphore()
        for s in range(1, N_DEV):
            dst = lax.rem(my_i + s, N_DEV)
            pl.semaphore_signal(
                barrier, inc=1,
                device_id=(dst,), device_id_type=pl.DeviceIdType.MESH,
            )

        x_dma.wait()
        x_bf[...] = x_st[...].astype(jnp.bfloat16)
        xrot[:, pl.ds(0, k_sh)] = x_bf[pl.ds(my_i * m_per, m_per), :]

        pl.semaphore_wait(barrier, N_DEV - 1)

        sends = []
        for s in range(1, N_DEV):
            dst = lax.rem(my_i + s, N_DEV)
            rel = N_DEV - s
            rdma = pltpu.make_async_remote_copy(
                src_ref=x_bf.at[pl.ds(dst * m_per, m_per), :],
                dst_ref=xrot.at[:, pl.ds(rel * k_sh, k_sh)],
                send_sem=send_sems.at[s - 1],
                recv_sem=recv_sems.at[rel],
                device_id=(dst,),
                device_id_type=pl.DeviceIdType.MESH,
            )
            rdma.start()
            sends.append(rdma)

        for k in range(N_DEV):
            w_dma(k).start()

        for k in range(N_DEV):
            w_dma(k).wait()
            if k >= 1:
                s = k
                rel = N_DEV - s
                recv = pltpu.make_async_remote_copy(
                    src_ref=x_bf.at[pl.ds(0, m_per), :],
                    dst_ref=xrot.at[:, pl.ds(rel * k_sh, k_sh)],
                    send_sem=send_sems.at[s - 1],
                    recv_sem=recv_sems.at[rel],
                    device_id=(lax.rem(my_i - s + N_DEV, N_DEV),),
                    device_id_type=pl.DeviceIdType.MESH,
                )
                recv.wait_recv()
            if k in _GROUPS:
                r0, r1 = _GROUPS[k]
                c0, c1 = r0 * k_sh, r1 * k_sh
                part = jnp.dot(
                    xrot[:, c0:c1],
                    w_st[c0:c1, :].astype(jnp.bfloat16),
                    preferred_element_type=jnp.float32,
                )
                if k == 0:
                    acc[...] = part
                else:
                    acc[...] += part

        for rdma in sends:
            rdma.wait_send()

        y = acc[...]
        c = 0.7978845608028654
        out_ref[...] = (
            0.5 * y * (1.0 + jnp.tanh(c * (y + 0.044715 * y * y * y)))
        ).astype(jnp.bfloat16)

    return pl.pallas_call(
        body,
        out_shape=jax.ShapeDtypeStruct((m_per, n), jnp.bfloat16),
        in_specs=[
            pl.BlockSpec(memory_space=pl.ANY),
            pl.BlockSpec(memory_space=pl.ANY),
        ],
        out_specs=pl.BlockSpec(memory_space=pltpu.VMEM),
        scratch_shapes=[
            pltpu.VMEM((m_total, k_sh), jnp.float32),
            pltpu.VMEM((m_total, k_sh), jnp.bfloat16),
            pltpu.VMEM((m_per, k_total), jnp.bfloat16),
            pltpu.VMEM((k_total, n), jnp.float32),
            pltpu.VMEM((m_per, n), jnp.float32),
            pltpu.SemaphoreType.DMA((N_DEV - 1,)),
            pltpu.SemaphoreType.DMA((N_DEV,)),
            pltpu.SemaphoreType.DMA,
            pltpu.SemaphoreType.DMA((N_DEV,)),
        ],
        compiler_params=pltpu.CompilerParams(
            collective_id=0,
            vmem_limit_bytes=100 * 1024 * 1024,
        ),
    )(x, w_mat)
